# baseline (speedup 1.0000x reference)
import jax
import jax.numpy as jnp
from jax import lax
from jax.experimental import pallas as pl
from jax.experimental.pallas import tpu as pltpu
from jax.experimental.pallas import tpu_sc as plsc

NUM_CORES = 2
NUM_SUBCORES = 16
LANES = 16
NW = NUM_CORES * NUM_SUBCORES

NUM_ROWS = 1000000
BATCH = 16384
LATENT = 16
B_PER_W = BATCH // NW
CHUNK = 128
NCHUNK = B_PER_W // CHUNK
SUB = CHUNK // LANES
DBLK_STRIDE = ((NUM_ROWS + 127) // 128) * 1024


def _body(users_ref, items_ref, utab_ref, itab_ref, out_ref,
          idx_u, idx_i, gidx_u, gidx_i, buf_u, buf_i, out_v, sem_u, sem_i):
    wid = lax.axis_index("s") * NUM_CORES + lax.axis_index("c")
    base = wid * B_PER_W

    pltpu.sync_copy(users_ref.at[pl.ds(base, B_PER_W)], idx_u)
    pltpu.sync_copy(items_ref.at[pl.ds(base, B_PER_W)], idx_i)

    def build(c, _):
        for t in range(SUB):
            sl = pl.ds(c * CHUNK + t * LANES, LANES)
            tsl = pl.ds(t * LANES, LANES)
            gidx_u[c, tsl] = idx_u[sl]
            gidx_i[c, tsl] = idx_i[sl]
        return 0

    lax.fori_loop(0, NCHUNK, build, 0)

    copies = []
    for d in range(LATENT):
        for c in range(NCHUNK):
            sl = pl.ds(c * CHUNK, CHUNK)
            cu = pltpu.make_async_copy(
                utab_ref.at[d].at[gidx_u.at[c]], buf_u.at[d].at[sl], sem_u)
            ci = pltpu.make_async_copy(
                itab_ref.at[d].at[gidx_i.at[c]], buf_i.at[d].at[sl], sem_i)
            cu.start()
            ci.start()
            copies.append(cu)
            copies.append(ci)
    for cp in copies:
        cp.wait()

    def group(g, _):
        sl = pl.ds(g * LANES, LANES)
        acc = jnp.zeros((LANES,), jnp.float32)
        for d in range(LATENT):
            acc = acc + buf_u[d, sl] * buf_i[d, sl]
        out_v[sl] = 1.0 / (1.0 + jnp.exp(-acc))
        return 0

    lax.fori_loop(0, B_PER_W // LANES, group, 0)

    pltpu.sync_copy(out_v, out_ref.at[pl.ds(base, B_PER_W)])


@jax.jit
def kernel(users, items, user_table, item_table):
    ut2 = user_table.T
    it2 = item_table.T
    mesh = plsc.VectorSubcoreMesh(
        core_axis_name="c", subcore_axis_name="s",
        num_cores=NUM_CORES, num_subcores=NUM_SUBCORES)
    run = pl.kernel(
        _body,
        out_type=jax.ShapeDtypeStruct((BATCH,), jnp.float32),
        mesh=mesh,
        scratch_types=[
            pltpu.VMEM((B_PER_W,), jnp.int32),
            pltpu.VMEM((B_PER_W,), jnp.int32),
            pltpu.VMEM((NCHUNK, CHUNK), jnp.int32),
            pltpu.VMEM((NCHUNK, CHUNK), jnp.int32),
            pltpu.VMEM((LATENT, B_PER_W), jnp.float32),
            pltpu.VMEM((LATENT, B_PER_W), jnp.float32),
            pltpu.VMEM((B_PER_W,), jnp.float32),
            pltpu.SemaphoreType.DMA,
            pltpu.SemaphoreType.DMA,
        ],
        compiler_params=pltpu.CompilerParams(
            needs_layout_passes=False,
            use_tc_tiling_on_sc=False,
        ),
    )
    return run(users, items, ut2, it2)
